# baseline (speedup 1.0000x reference)
import functools

import jax
import jax.numpy as jnp
from jax.experimental import pallas as pl
from jax.experimental.pallas import tpu as pltpu

NUM_QUERIES = 300
NUM_CLASSES = 10
TOPK = 100

_FLAT = NUM_QUERIES * NUM_CLASSES
_PAD_ROWS = 24


def _topk_kernel(probs_ref, vals_ref, qidx_ref):
    x = probs_ref[...]
    r24 = jax.lax.broadcasted_iota(jnp.int32, (_PAD_ROWS, 128), 0)
    c24 = jax.lax.broadcasted_iota(jnp.int32, (_PAD_ROWS, 128), 1)
    flat = r24 * 128 + c24
    r8 = jax.lax.broadcasted_iota(jnp.int32, (8, 128), 0)
    c8 = jax.lax.broadcasted_iota(jnp.int32, (8, 128), 1)

    def body(k, carry):
        x, vacc, iacc = carry
        m = jnp.max(x)
        chosen = jnp.min(jnp.where(x == m, flat, jnp.int32(1 << 30)))
        x = jnp.where(flat == chosen, jnp.float32(-1.0), x)
        sel = (r8 == 0) & (c8 == k)
        vacc = jnp.where(sel, m, vacc)
        iacc = jnp.where(sel, chosen // NUM_CLASSES, iacc)
        return x, vacc, iacc

    _, vacc, iacc = jax.lax.fori_loop(
        0, TOPK, body,
        (x, jnp.zeros((8, 128), jnp.float32), jnp.zeros((8, 128), jnp.int32)),
    )
    vals_ref[...] = vacc
    qidx_ref[...] = iacc


def _mask_kernel(qidx_ref, masks_ref, boxes_ref, vals_ref,
                 mout_ref, lab_ref, boxout_ref):
    k = pl.program_id(0)
    x = masks_ref[0]
    pos = x > 0
    binf = jnp.where(pos, jnp.float32(1.0), jnp.float32(0.0))
    mout_ref[0] = binf
    th = jnp.tanh(x * 0.5)
    tsum = jnp.sum(jnp.where(pos, th, jnp.float32(0.0)))
    bsum = jnp.sum(binf)
    conf = (0.5 * bsum + 0.5 * tsum) / (bsum + 1e-6)

    r8 = jax.lax.broadcasted_iota(jnp.int32, (8, 128), 0)
    c8 = jax.lax.broadcasted_iota(jnp.int32, (8, 128), 1)
    sel = (r8 == 0) & (c8 == k)

    @pl.when(k == 0)
    def _():
        lab_ref[...] = jnp.zeros_like(lab_ref)

    lab_ref[...] = jnp.where(sel, conf, lab_ref[...])

    @pl.when(k == TOPK - 1)
    def _():
        lab_ref[...] = lab_ref[...] * vals_ref[...]

    q = qidx_ref[k]
    boxout_ref[pl.ds(k, 1), :] = boxes_ref[pl.ds(q, 1), :]


def kernel(predicted_labels, predicted_masks, predicted_boxes):
    probs = jax.nn.sigmoid(predicted_labels)
    flat = probs.reshape(-1)
    padded = jnp.concatenate(
        [flat, jnp.full((_PAD_ROWS * 128 - _FLAT,), -1.0, jnp.float32)]
    ).reshape(_PAD_ROWS, 128)

    vals8, qidx8 = pl.pallas_call(
        _topk_kernel,
        out_shape=[
            jax.ShapeDtypeStruct((8, 128), jnp.float32),
            jax.ShapeDtypeStruct((8, 128), jnp.int32),
        ],
    )(padded)

    qidx = qidx8[0, :TOPK]

    grid_spec = pltpu.PrefetchScalarGridSpec(
        num_scalar_prefetch=1,
        grid=(TOPK,),
        in_specs=[
            pl.BlockSpec((1, 16, 96, 96), lambda k, idx: (idx[k], 0, 0, 0)),
            pl.BlockSpec((NUM_QUERIES, 6), lambda k, idx: (0, 0)),
            pl.BlockSpec((8, 128), lambda k, idx: (0, 0)),
        ],
        out_specs=[
            pl.BlockSpec((1, 16, 96, 96), lambda k, idx: (k, 0, 0, 0)),
            pl.BlockSpec((8, 128), lambda k, idx: (0, 0)),
            pl.BlockSpec((TOPK, 6), lambda k, idx: (0, 0)),
        ],
    )
    mout, lab8, boxes_sel = pl.pallas_call(
        _mask_kernel,
        grid_spec=grid_spec,
        out_shape=[
            jax.ShapeDtypeStruct((TOPK, 16, 96, 96), jnp.float32),
            jax.ShapeDtypeStruct((8, 128), jnp.float32),
            jax.ShapeDtypeStruct((TOPK, 6), jnp.float32),
        ],
    )(qidx, predicted_masks, predicted_boxes, vals8)

    labels_out = lab8[0, :TOPK]
    return (labels_out, boxes_sel, mout)


# speedup vs baseline: 1.1074x; 1.1074x over previous
import functools

import jax
import jax.numpy as jnp
from jax.experimental import pallas as pl
from jax.experimental.pallas import tpu as pltpu

NUM_QUERIES = 300
NUM_CLASSES = 10
TOPK = 100

_FLAT = NUM_QUERIES * NUM_CLASSES
_PAD_ROWS = 24
_D = 4


def _topk_kernel(probs_ref, vals_ref, qidx_ref):
    x = probs_ref[...]
    r24 = jax.lax.broadcasted_iota(jnp.int32, (_PAD_ROWS, 128), 0)
    c24 = jax.lax.broadcasted_iota(jnp.int32, (_PAD_ROWS, 128), 1)
    flat = r24 * 128 + c24
    r8 = jax.lax.broadcasted_iota(jnp.int32, (8, 128), 0)
    c8 = jax.lax.broadcasted_iota(jnp.int32, (8, 128), 1)

    def body(k, carry):
        x, vacc, iacc = carry
        m = jnp.max(x)
        chosen = jnp.min(jnp.where(x == m, flat, jnp.int32(1 << 30)))
        x = jnp.where(flat == chosen, jnp.float32(-1.0), x)
        sel = (r8 == 0) & (c8 == k)
        vacc = jnp.where(sel, m, vacc)
        iacc = jnp.where(sel, chosen // NUM_CLASSES, iacc)
        return x, vacc, iacc

    _, vacc, iacc = jax.lax.fori_loop(
        0, TOPK, body,
        (x, jnp.zeros((8, 128), jnp.float32), jnp.zeros((8, 128), jnp.int32)),
    )
    vals_ref[...] = vacc
    qidx_ref[...] = iacc


def _mask_kernel(qidx_ref, masks_ref, boxes_ref, vals_ref,
                 mout_ref, lab_ref, boxout_ref,
                 inbuf, outbuf, insems, outsems):
    def in_copy(k, slot):
        q = qidx_ref[k]
        return pltpu.make_async_copy(
            masks_ref.at[pl.ds(q, 1)], inbuf.at[pl.ds(slot, 1)],
            insems.at[slot])

    def out_copy(k, slot):
        return pltpu.make_async_copy(
            outbuf.at[pl.ds(slot, 1)], mout_ref.at[pl.ds(k, 1)],
            outsems.at[slot])

    for k0 in range(_D):
        in_copy(k0, k0).start()

    r8 = jax.lax.broadcasted_iota(jnp.int32, (8, 128), 0)
    c8 = jax.lax.broadcasted_iota(jnp.int32, (8, 128), 1)

    def step(k, conf_acc):
        slot = jax.lax.rem(k, _D)
        in_copy(k, slot).wait()
        x = inbuf[slot]
        pos = x > 0
        binf = jnp.where(pos, jnp.float32(1.0), jnp.float32(0.0))
        th = jnp.tanh(x * 0.5)
        tsum = jnp.sum(jnp.where(pos, th, jnp.float32(0.0)))
        bsum = jnp.sum(binf)
        conf = (0.5 * bsum + 0.5 * tsum) / (bsum + 1e-6)

        @pl.when(k >= _D)
        def _():
            out_copy(k - _D, slot).wait()

        outbuf[pl.ds(slot, 1)] = binf[None]
        out_copy(k, slot).start()

        @pl.when(k + _D < TOPK)
        def _():
            in_copy(k + _D, slot).start()

        q = qidx_ref[k]
        boxout_ref[pl.ds(k, 1), :] = boxes_ref[pl.ds(q, 1), :]

        sel = (r8 == 0) & (c8 == k)
        return jnp.where(sel, conf, conf_acc)

    conf_acc = jax.lax.fori_loop(
        0, TOPK, step, jnp.zeros((8, 128), jnp.float32))

    for t in range(_D):
        k = TOPK - _D + t
        out_copy(k, k % _D).wait()

    lab_ref[...] = conf_acc * vals_ref[...]


def kernel(predicted_labels, predicted_masks, predicted_boxes):
    probs = jax.nn.sigmoid(predicted_labels)
    flat = probs.reshape(-1)
    padded = jnp.concatenate(
        [flat, jnp.full((_PAD_ROWS * 128 - _FLAT,), -1.0, jnp.float32)]
    ).reshape(_PAD_ROWS, 128)

    vals8, qidx8 = pl.pallas_call(
        _topk_kernel,
        out_shape=[
            jax.ShapeDtypeStruct((8, 128), jnp.float32),
            jax.ShapeDtypeStruct((8, 128), jnp.int32),
        ],
    )(padded)

    qidx = qidx8[0, :TOPK]

    grid_spec = pltpu.PrefetchScalarGridSpec(
        num_scalar_prefetch=1,
        grid=(1,),
        in_specs=[
            pl.BlockSpec(memory_space=pltpu.MemorySpace.HBM),
            pl.BlockSpec((NUM_QUERIES, 6), lambda i, idx: (0, 0)),
            pl.BlockSpec((8, 128), lambda i, idx: (0, 0)),
        ],
        out_specs=[
            pl.BlockSpec(memory_space=pltpu.MemorySpace.HBM),
            pl.BlockSpec((8, 128), lambda i, idx: (0, 0)),
            pl.BlockSpec((TOPK, 6), lambda i, idx: (0, 0)),
        ],
        scratch_shapes=[
            pltpu.VMEM((_D, 16, 96, 96), jnp.float32),
            pltpu.VMEM((_D, 16, 96, 96), jnp.float32),
            pltpu.SemaphoreType.DMA((_D,)),
            pltpu.SemaphoreType.DMA((_D,)),
        ],
    )
    mout, lab8, boxes_sel = pl.pallas_call(
        _mask_kernel,
        grid_spec=grid_spec,
        out_shape=[
            jax.ShapeDtypeStruct((TOPK, 16, 96, 96), jnp.float32),
            jax.ShapeDtypeStruct((8, 128), jnp.float32),
            jax.ShapeDtypeStruct((TOPK, 6), jnp.float32),
        ],
    )(qidx, predicted_masks, predicted_boxes, vals8)

    labels_out = lab8[0, :TOPK]
    return (labels_out, boxes_sel, mout)


# speedup vs baseline: 1.8507x; 1.6713x over previous
import functools

import jax
import jax.numpy as jnp
from jax.experimental import pallas as pl
from jax.experimental.pallas import tpu as pltpu

NUM_QUERIES = 300
NUM_CLASSES = 10
TOPK = 100

_FLAT = NUM_QUERIES * NUM_CLASSES
_PAD_ROWS = 24
_M = 16 * 96 * 96
_BM = 1536
_STEPS = _M // _BM


def _topk_kernel(probs_ref, vals_ref, onehot_ref):
    x = probs_ref[...]
    r24 = jax.lax.broadcasted_iota(jnp.int32, (_PAD_ROWS, 128), 0)
    c24 = jax.lax.broadcasted_iota(jnp.int32, (_PAD_ROWS, 128), 1)
    flat = r24 * 128 + c24
    r8 = jax.lax.broadcasted_iota(jnp.int32, (8, 128), 0)
    c8 = jax.lax.broadcasted_iota(jnp.int32, (8, 128), 1)

    def body(k, carry):
        x, vacc, iacc = carry
        m = jnp.max(x)
        chosen = jnp.min(jnp.where(x == m, flat, jnp.int32(1 << 30)))
        x = jnp.where(flat == chosen, jnp.float32(-1.0), x)
        sel = (r8 == 0) & (c8 == k)
        vacc = jnp.where(sel, m, vacc)
        iacc = jnp.where(sel, chosen // NUM_CLASSES, iacc)
        return x, vacc, iacc

    _, vacc, iacc = jax.lax.fori_loop(
        0, TOPK, body,
        (x, jnp.zeros((8, 128), jnp.float32), jnp.zeros((8, 128), jnp.int32)),
    )
    vals_ref[...] = vacc
    qrow = iacc[0:1, :]
    riota = jax.lax.broadcasted_iota(jnp.int32, (NUM_QUERIES, 128), 0)
    onehot_ref[...] = jnp.where(
        riota == qrow, jnp.float32(1.0), jnp.float32(0.0))


def _mask_kernel(masks_ref, onehot_ref, boxes_ref, vals_ref,
                 binout_ref, misc_ref, boxout_ref):
    i = pl.program_id(0)
    w = onehot_ref[...]
    x = masks_ref[...]
    g = jax.lax.dot_general(
        x, w, (((1,), (0,)), ((), ())),
        precision=jax.lax.Precision.HIGHEST,
        preferred_element_type=jnp.float32)
    pos = g > 0
    binf = jnp.where(pos, jnp.float32(1.0), jnp.float32(0.0))
    binout_ref[...] = binf[:, :TOPK]
    th = jnp.tanh(g * 0.5)
    tpart = jnp.sum(jnp.where(pos, th, jnp.float32(0.0)), axis=0,
                    keepdims=True)
    bpart = jnp.sum(binf, axis=0, keepdims=True)

    @pl.when(i == 0)
    def _():
        misc_ref[...] = jnp.zeros_like(misc_ref)
        bg = jax.lax.dot_general(
            boxes_ref[...], w, (((1,), (0,)), ((), ())),
            precision=jax.lax.Precision.HIGHEST,
            preferred_element_type=jnp.float32)
        boxout_ref[...] = bg[:, :TOPK]

    misc_ref[0:1, :] = misc_ref[0:1, :] + tpart
    misc_ref[1:2, :] = misc_ref[1:2, :] + bpart

    @pl.when(i == _STEPS - 1)
    def _():
        t = misc_ref[0:1, :]
        b = misc_ref[1:2, :]
        conf = (0.5 * b + 0.5 * t) / (b + 1e-6)
        misc_ref[2:3, :] = vals_ref[0:1, :] * conf


def kernel(predicted_labels, predicted_masks, predicted_boxes):
    probs = jax.nn.sigmoid(predicted_labels)
    flat = probs.reshape(-1)
    padded = jnp.concatenate(
        [flat, jnp.full((_PAD_ROWS * 128 - _FLAT,), -1.0, jnp.float32)]
    ).reshape(_PAD_ROWS, 128)

    vals8, onehot = pl.pallas_call(
        _topk_kernel,
        out_shape=[
            jax.ShapeDtypeStruct((8, 128), jnp.float32),
            jax.ShapeDtypeStruct((NUM_QUERIES, 128), jnp.float32),
        ],
    )(padded)

    masks2 = predicted_masks.transpose(1, 2, 3, 0).reshape(_M, NUM_QUERIES)
    boxes_t = predicted_boxes.transpose(1, 0)

    binout, misc, boxout = pl.pallas_call(
        _mask_kernel,
        grid=(_STEPS,),
        in_specs=[
            pl.BlockSpec((_BM, NUM_QUERIES), lambda i: (i, 0)),
            pl.BlockSpec((NUM_QUERIES, 128), lambda i: (0, 0)),
            pl.BlockSpec((6, NUM_QUERIES), lambda i: (0, 0)),
            pl.BlockSpec((8, 128), lambda i: (0, 0)),
        ],
        out_specs=[
            pl.BlockSpec((_BM, TOPK), lambda i: (i, 0)),
            pl.BlockSpec((8, 128), lambda i: (0, 0)),
            pl.BlockSpec((6, TOPK), lambda i: (0, 0)),
        ],
        out_shape=[
            jax.ShapeDtypeStruct((_M, TOPK), jnp.float32),
            jax.ShapeDtypeStruct((8, 128), jnp.float32),
            jax.ShapeDtypeStruct((6, TOPK), jnp.float32),
        ],
    )(masks2, onehot, boxes_t, vals8)

    labels_out = misc[2, :TOPK]
    boxes_sel = boxout.transpose(1, 0)
    masks_bin = binout.reshape(16, 96, 96, TOPK).transpose(3, 0, 1, 2)
    return (labels_out, boxes_sel, masks_bin)


# speedup vs baseline: 2.5773x; 1.3926x over previous
import functools

import jax
import jax.numpy as jnp
from jax.experimental import pallas as pl
from jax.experimental.pallas import tpu as pltpu

NUM_QUERIES = 300
NUM_CLASSES = 10
TOPK = 100

_FLAT = NUM_QUERIES * NUM_CLASSES
_PAD_ROWS = 24
_M = 16 * 96 * 96
_BM = 1536
_STEPS = _M // _BM


def _topk_kernel(probs_ref, vals_ref, onehot_ref):
    x = probs_ref[...]
    r24 = jax.lax.broadcasted_iota(jnp.int32, (_PAD_ROWS, 128), 0)
    c24 = jax.lax.broadcasted_iota(jnp.int32, (_PAD_ROWS, 128), 1)
    flat = r24 * 128 + c24
    r8 = jax.lax.broadcasted_iota(jnp.int32, (8, 128), 0)
    c8 = jax.lax.broadcasted_iota(jnp.int32, (8, 128), 1)

    def body(k, carry):
        x, vacc, iacc = carry
        m = jnp.max(x)
        chosen = jnp.min(jnp.where(x == m, flat, jnp.int32(1 << 30)))
        x = jnp.where(flat == chosen, jnp.float32(-1.0), x)
        sel = (r8 == 0) & (c8 == k)
        vacc = jnp.where(sel, m, vacc)
        iacc = jnp.where(sel, chosen // NUM_CLASSES, iacc)
        return x, vacc, iacc

    _, vacc, iacc = jax.lax.fori_loop(
        0, TOPK, body,
        (x, jnp.zeros((8, 128), jnp.float32), jnp.zeros((8, 128), jnp.int32)),
    )
    vals_ref[...] = vacc
    qrow = iacc[0:1, :]
    riota = jax.lax.broadcasted_iota(jnp.int32, (NUM_QUERIES, 128), 0)
    onehot_ref[...] = jnp.where(
        riota == qrow, jnp.float32(1.0), jnp.float32(0.0))


def _mask_kernel(masks_ref, onehot_ref, boxes_ref, vals_ref,
                 binout_ref, misc_ref, boxout_ref):
    i = pl.program_id(0)
    w = onehot_ref[...]
    x = masks_ref[...]
    sgn = jnp.where(x > 0, jnp.float32(1.0), jnp.float32(0.0))
    binf = jax.lax.dot_general(
        sgn, w, (((1,), (0,)), ((), ())),
        preferred_element_type=jnp.float32)
    binout_ref[...] = binf[:, :TOPK]
    g = jax.lax.dot_general(
        x, w, (((1,), (0,)), ((), ())),
        preferred_element_type=jnp.float32)
    th = jnp.tanh(g * 0.5)
    tpart = jnp.sum(th * binf, axis=0, keepdims=True)
    bpart = jnp.sum(binf, axis=0, keepdims=True)

    @pl.when(i == 0)
    def _():
        misc_ref[...] = jnp.zeros_like(misc_ref)
        bg = jax.lax.dot_general(
            boxes_ref[...], w, (((1,), (0,)), ((), ())),
            precision=jax.lax.Precision.HIGHEST,
            preferred_element_type=jnp.float32)
        boxout_ref[...] = bg[:, :TOPK]

    misc_ref[0:1, :] = misc_ref[0:1, :] + tpart
    misc_ref[1:2, :] = misc_ref[1:2, :] + bpart

    @pl.when(i == _STEPS - 1)
    def _():
        t = misc_ref[0:1, :]
        b = misc_ref[1:2, :]
        conf = (0.5 * b + 0.5 * t) / (b + 1e-6)
        misc_ref[2:3, :] = vals_ref[0:1, :] * conf


def kernel(predicted_labels, predicted_masks, predicted_boxes):
    probs = jax.nn.sigmoid(predicted_labels)
    flat = probs.reshape(-1)
    padded = jnp.concatenate(
        [flat, jnp.full((_PAD_ROWS * 128 - _FLAT,), -1.0, jnp.float32)]
    ).reshape(_PAD_ROWS, 128)

    vals8, onehot = pl.pallas_call(
        _topk_kernel,
        out_shape=[
            jax.ShapeDtypeStruct((8, 128), jnp.float32),
            jax.ShapeDtypeStruct((NUM_QUERIES, 128), jnp.float32),
        ],
    )(padded)

    masks2 = predicted_masks.transpose(1, 2, 3, 0).reshape(_M, NUM_QUERIES)
    boxes_t = predicted_boxes.transpose(1, 0)

    binout, misc, boxout = pl.pallas_call(
        _mask_kernel,
        grid=(_STEPS,),
        in_specs=[
            pl.BlockSpec((_BM, NUM_QUERIES), lambda i: (i, 0)),
            pl.BlockSpec((NUM_QUERIES, 128), lambda i: (0, 0)),
            pl.BlockSpec((6, NUM_QUERIES), lambda i: (0, 0)),
            pl.BlockSpec((8, 128), lambda i: (0, 0)),
        ],
        out_specs=[
            pl.BlockSpec((_BM, TOPK), lambda i: (i, 0)),
            pl.BlockSpec((8, 128), lambda i: (0, 0)),
            pl.BlockSpec((6, TOPK), lambda i: (0, 0)),
        ],
        out_shape=[
            jax.ShapeDtypeStruct((_M, TOPK), jnp.float32),
            jax.ShapeDtypeStruct((8, 128), jnp.float32),
            jax.ShapeDtypeStruct((6, TOPK), jnp.float32),
        ],
    )(masks2, onehot, boxes_t, vals8)

    labels_out = misc[2, :TOPK]
    boxes_sel = boxout.transpose(1, 0)
    masks_bin = binout.reshape(16, 96, 96, TOPK).transpose(3, 0, 1, 2)
    return (labels_out, boxes_sel, masks_bin)


# speedup vs baseline: 3.7482x; 1.4543x over previous
import functools

import jax
import jax.numpy as jnp
from jax.experimental import pallas as pl
from jax.experimental.pallas import tpu as pltpu

NUM_QUERIES = 300
NUM_CLASSES = 10
TOPK = 100

_FLAT = NUM_QUERIES * NUM_CLASSES
_PAD_ROWS = 24
_M = 16 * 96 * 96
_BM = 1536
_STEPS = _M // _BM
_D = 12


def _fused_kernel(probs_ref, masks_ref, boxes_ref,
                  binout_ref, misc_ref, boxout_ref,
                  onehot_s, vals_s, inbuf, sems):
    i = pl.program_id(0)

    def in_copy(step, slot):
        return pltpu.make_async_copy(
            masks_ref.at[pl.ds(step * _BM, _BM)],
            inbuf.at[slot],
            sems.at[slot])

    @pl.when(i == 0)
    def _():
        for s in range(_D):
            in_copy(s, s).start()

        x = probs_ref[...]
        r24 = jax.lax.broadcasted_iota(jnp.int32, (_PAD_ROWS, 128), 0)
        c24 = jax.lax.broadcasted_iota(jnp.int32, (_PAD_ROWS, 128), 1)
        flat = r24 * 128 + c24
        r8 = jax.lax.broadcasted_iota(jnp.int32, (8, 128), 0)
        c8 = jax.lax.broadcasted_iota(jnp.int32, (8, 128), 1)

        def body(k, carry):
            x, vacc, iacc = carry
            m = jnp.max(x)
            chosen = jnp.min(jnp.where(x == m, flat, jnp.int32(1 << 30)))
            x = jnp.where(flat == chosen, jnp.float32(-1.0), x)
            sel = (r8 == 0) & (c8 == k)
            vacc = jnp.where(sel, m, vacc)
            iacc = jnp.where(sel, chosen // NUM_CLASSES, iacc)
            return x, vacc, iacc

        _, vacc, iacc = jax.lax.fori_loop(
            0, TOPK, body,
            (x, jnp.zeros((8, 128), jnp.float32),
             jnp.zeros((8, 128), jnp.int32)),
        )
        vals_s[...] = vacc
        qrow = iacc[0:1, :]
        riota = jax.lax.broadcasted_iota(jnp.int32, (NUM_QUERIES, 128), 0)
        w0 = jnp.where(riota == qrow, jnp.float32(1.0), jnp.float32(0.0))
        onehot_s[...] = w0

        misc_ref[...] = jnp.zeros_like(misc_ref)
        bg = jax.lax.dot_general(
            boxes_ref[...], w0, (((1,), (0,)), ((), ())),
            precision=jax.lax.Precision.HIGHEST,
            preferred_element_type=jnp.float32)
        boxout_ref[...] = bg[:, :TOPK]

    slot = jax.lax.rem(i, _D)
    in_copy(i, slot).wait()
    x = inbuf[slot]
    w = onehot_s[...]
    sgn = jnp.where(x > 0, jnp.float32(1.0), jnp.float32(0.0))
    binf = jax.lax.dot_general(
        sgn, w, (((1,), (0,)), ((), ())),
        preferred_element_type=jnp.float32)
    binout_ref[...] = binf[:, :TOPK]
    g = jax.lax.dot_general(
        x, w, (((1,), (0,)), ((), ())),
        preferred_element_type=jnp.float32)
    th = jnp.tanh(g * 0.5)
    tpart = jnp.sum(th * binf, axis=0, keepdims=True)
    bpart = jnp.sum(binf, axis=0, keepdims=True)

    misc_ref[0:1, :] = misc_ref[0:1, :] + tpart
    misc_ref[1:2, :] = misc_ref[1:2, :] + bpart

    @pl.when(i + _D < _STEPS)
    def _():
        in_copy(i + _D, slot).start()

    @pl.when(i == _STEPS - 1)
    def _():
        t = misc_ref[0:1, :]
        b = misc_ref[1:2, :]
        conf = (0.5 * b + 0.5 * t) / (b + 1e-6)
        misc_ref[2:3, :] = vals_s[0:1, :] * conf


def kernel(predicted_labels, predicted_masks, predicted_boxes):
    probs = jax.nn.sigmoid(predicted_labels)
    flat = probs.reshape(-1)
    padded = jnp.concatenate(
        [flat, jnp.full((_PAD_ROWS * 128 - _FLAT,), -1.0, jnp.float32)]
    ).reshape(_PAD_ROWS, 128)

    masks2 = predicted_masks.transpose(1, 2, 3, 0).reshape(_M, NUM_QUERIES)
    boxes_t = predicted_boxes.transpose(1, 0)

    binout, misc, boxout = pl.pallas_call(
        _fused_kernel,
        grid=(_STEPS,),
        in_specs=[
            pl.BlockSpec((_PAD_ROWS, 128), lambda i: (0, 0)),
            pl.BlockSpec(memory_space=pltpu.MemorySpace.HBM),
            pl.BlockSpec((6, NUM_QUERIES), lambda i: (0, 0)),
        ],
        out_specs=[
            pl.BlockSpec((_BM, TOPK), lambda i: (i, 0)),
            pl.BlockSpec((8, 128), lambda i: (0, 0)),
            pl.BlockSpec((6, TOPK), lambda i: (0, 0)),
        ],
        out_shape=[
            jax.ShapeDtypeStruct((_M, TOPK), jnp.float32),
            jax.ShapeDtypeStruct((8, 128), jnp.float32),
            jax.ShapeDtypeStruct((6, TOPK), jnp.float32),
        ],
        scratch_shapes=[
            pltpu.VMEM((NUM_QUERIES, 128), jnp.float32),
            pltpu.VMEM((8, 128), jnp.float32),
            pltpu.VMEM((_D, _BM, NUM_QUERIES), jnp.float32),
            pltpu.SemaphoreType.DMA((_D,)),
        ],
    )(padded, masks2, boxes_t)

    labels_out = misc[2, :TOPK]
    boxes_sel = boxout.transpose(1, 0)
    masks_bin = binout.reshape(16, 96, 96, TOPK).transpose(3, 0, 1, 2)
    return (labels_out, boxes_sel, masks_bin)
